# Initial kernel scaffold; baseline (speedup 1.0000x reference)
import jax
import jax.numpy as jnp
from jax.experimental import pallas as pl


def kernel(x):
    raise NotImplementedError("write your pallas kernel here")


# speedup vs baseline: 5.6353x; 5.6353x over previous
import functools

import jax
import jax.numpy as jnp
import numpy as np
from jax import lax
from jax.experimental import pallas as pl
from jax.experimental.pallas import tpu as pltpu
from jax.experimental.pallas import tpu_sc as plsc

_T = 16384
_LM = 115
_W = 345
_BF = 512
_NOUT = 128


_HI, _HJ = np.triu_indices(21, k=1)
_PI, _PJ = np.triu_indices(25, k=1)
_LI, _LJ = np.triu_indices(20, k=1)


def _build_consts():
    s1 = np.zeros((258, 153), np.float32)
    for j in range(63):
        s1[j, j] = 1.0
    for q in range(50):
        s1[3 * (21 + q // 2) + q % 2, 63 + q] = 1.0
    for q in range(40):
        s1[3 * (46 + q // 2) + q % 2, 113 + q] = 1.0

    g = np.zeros((3, 258, 890), np.float32)
    for p, (a, b) in enumerate(zip(_HI, _HJ)):
        for c in range(3):
            g[c, 3 * a + c, p] += 1.0
            g[c, 3 * b + c, p] -= 1.0
    for p, (a, b) in enumerate(zip(_PI, _PJ)):
        for c in range(2):
            g[c, 3 * (21 + a) + c, 210 + p] += 1.0
            g[c, 3 * (21 + b) + c, 210 + p] -= 1.0
    for p, (a, b) in enumerate(zip(_LI, _LJ)):
        for c in range(2):
            g[c, 3 * (46 + a) + c, 510 + p] += 1.0
            g[c, 3 * (46 + b) + c, 510 + p] -= 1.0
    for p, (a, b) in enumerate(zip(_LI, _LJ)):
        for c in range(2):
            g[c, 3 * (66 + a) + c, 700 + p] += 1.0
            g[c, 3 * (66 + b) + c, 700 + p] -= 1.0
    return s1, g[0], g[1], g[2]


_S1, _G0, _G1, _G2 = _build_consts()


def _scan_body(x_ref, sums_ref, scal_ref):
    i = pl.program_id(0)
    xb = x_ref[...]
    xb = jnp.where(jnp.isnan(xb), jnp.float32(0.0), xb)
    col = lax.broadcasted_iota(jnp.int32, xb.shape, 1)
    lmask = (col >= 120) & (col < 183)
    rmask = col >= 282
    xl = jnp.where(lmask, xb, 0.0)
    xr = jnp.where(rmask, xb, 0.0)
    lsum = jnp.sum(xl, axis=1, keepdims=True)
    rsum = jnp.sum(xr, axis=1, keepdims=True)
    sums_ref[...] = jnp.concatenate([lsum, rsum], axis=1)
    lnz = jnp.sum(jnp.where(lmask & (xb != 0.0), 1.0, 0.0))
    rnz = jnp.sum(jnp.where(rmask & (xb != 0.0), 1.0, 0.0))
    nkl = jnp.sum(jnp.where(lsum != 0.0, 1.0, 0.0))
    nkr = jnp.sum(jnp.where(rsum != 0.0, 1.0, 0.0))
    ri = lax.broadcasted_iota(jnp.int32, (4, 16), 0)
    row = (jnp.where(ri == 0, lnz, 0.0) + jnp.where(ri == 1, rnz, 0.0)
           + jnp.where(ri == 2, nkl, 0.0) + jnp.where(ri == 3, nkr, 0.0))

    @pl.when(i == 0)
    def _():
        scal_ref[...] = jnp.zeros_like(scal_ref)

    scal_ref[...] += row


def _run_scan(x2d):
    return pl.pallas_call(
        _scan_body,
        grid=(_T // _BF,),
        in_specs=[pl.BlockSpec((_BF, _W), lambda i: (i, 0))],
        out_specs=[pl.BlockSpec((_BF, 2), lambda i: (i, 0)),
                   pl.BlockSpec((4, 16), lambda i: (0, 0))],
        out_shape=[jax.ShapeDtypeStruct((_T, 2), jnp.float32),
                   jax.ShapeDtypeStruct((4, 16), jnp.float32)],
    )(x2d)


@functools.cache
def _make_compact():
    mesh = plsc.VectorSubcoreMesh(core_axis_name="c", subcore_axis_name="s")
    return pl.kernel(
        _compact_body,
        mesh=mesh,
        compiler_params=pltpu.CompilerParams(needs_layout_passes=False),
        out_type=[jax.ShapeDtypeStruct((_NOUT,), jnp.int32),
                  jax.ShapeDtypeStruct((16,), jnp.int32)],
        scratch_types=[pltpu.VMEM((_T,), jnp.float32),
                       pltpu.VMEM((_T,), jnp.float32),
                       pltpu.VMEM((64,), jnp.float32),
                       pltpu.VMEM((16,), jnp.int32),
                       pltpu.VMEM((16,), jnp.int32),
                       pltpu.VMEM((16,), jnp.int32),
                       pltpu.VMEM((16,), jnp.int32),
                       pltpu.VMEM((16,), jnp.int32),
                       pltpu.VMEM((_NOUT,), jnp.int32),
                       pltpu.VMEM((16,), jnp.int32)],
    )


def _compact_body(lsum_hbm, rsum_hbm, scal_hbm, order_hbm, meta_hbm,
                  lv, rv, sv, tmp_v, cond_r, nk_r, fi_r, kc_r,
                  order_v, meta_v):
    first = (lax.axis_index("c") == 0) & (lax.axis_index("s") == 0)

    @pl.when(first)
    def _():
        pltpu.sync_copy(lsum_hbm, lv)
        pltpu.sync_copy(rsum_hbm, rv)
        pltpu.sync_copy(scal_hbm, sv)
        lane = lax.broadcasted_iota(jnp.int32, (16,), 0)
        lnzv = sv[pl.ds(0, 16)]
        rnzv = sv[pl.ds(16, 16)]
        cond_v = lnzv > rnzv
        nk_v = jnp.where(cond_v, sv[pl.ds(32, 16)],
                         sv[pl.ds(48, 16)]).astype(jnp.int32)
        neff_v = jnp.where(nk_v == 0, jnp.int32(_T), nk_v)
        cond_r[...] = cond_v.astype(jnp.int32)
        nk_r[...] = nk_v
        fi_r[...] = lane
        kc_r[...] = jnp.zeros((16,), jnp.int32)

        for j in range(_NOUT // 16):
            order_v[pl.ds(j * 16, 16)] = jnp.zeros((16,), jnp.int32)

        def body_fn(i, c):
            ln = lax.broadcasted_iota(jnp.int32, (16,), 0)
            condv = cond_r[...] != 0
            nkv = nk_r[...]
            fi = fi_r[...]
            kc = kc_r[...]
            lvv = lv[pl.ds(i * 16, 16)]
            rvv = rv[pl.ds(i * 16, 16)]
            selv = jnp.where(condv, lvv, rvv)
            kept = selv != 0.0
            ki = kept.astype(jnp.int32)
            cs = ki
            for k in (1, 2, 4, 8):
                tmp_v[...] = cs
                sh = plsc.load_gather(tmp_v, [jnp.maximum(ln - k, 0)],
                                      mask=ln >= k)
                cs = cs + jnp.where(ln >= k, sh, 0)
            kcv = kc + cs
            dest = jnp.where(kept, kcv - 1, nkv + fi - kcv)
            plsc.store_scatter(order_v, [dest], fi, mask=dest < _NOUT)
            tmp_v[...] = kcv
            kc_r[...] = plsc.load_gather(tmp_v, [jnp.full((16,), 15,
                                                          jnp.int32)])
            fi_r[...] = fi + 16
            return c

        lax.fori_loop(0, _T // 16, body_fn, jnp.int32(0))

        meta_v[...] = jnp.where(lane == 0, neff_v, 0)
        pltpu.sync_copy(order_v, order_hbm)
        pltpu.sync_copy(meta_v, meta_hbm)


def _gather_body(ord_ref, x_ref, o_ref):
    del ord_ref
    o_ref[...] = x_ref[...]


def _run_gather(order, x3):
    grid_spec = pltpu.PrefetchScalarGridSpec(
        num_scalar_prefetch=1,
        grid=(_NOUT,),
        in_specs=[pl.BlockSpec((1, 1, _W), lambda i, o: (o[i], 0, 0))],
        out_specs=pl.BlockSpec((1, 1, _W), lambda i, o: (i, 0, 0)),
    )
    return pl.pallas_call(
        _gather_body,
        grid_spec=grid_spec,
        out_shape=jax.ShapeDtypeStruct((_NOUT, 1, _W), jnp.float32),
    )(order, x3)


def _feat_body(xg_ref, scal_ref, meta_ref, s1_ref, g0_ref, g1_ref, g2_ref,
               o_ref):
    xg = xg_ref[...]
    xg = jnp.where(jnp.isnan(xg), jnp.float32(0.0), xg)
    cond = scal_ref[0, 0] > scal_ref[1, 0]
    neff = meta_ref[0, 0]
    hand = jnp.where(cond, xg[:, 120:183], xg[:, 282:345])
    xf = jnp.concatenate([hand, xg[:, 183:258], xg[:, 0:120]], axis=1)
    colmod = lax.broadcasted_iota(jnp.int32, (1, 258), 1) % 3
    xf = xf * jnp.where(cond & (colmod == 0), -1.0, 1.0)
    xf_next = jnp.concatenate(
        [xf[1:, :], jnp.zeros((1, 258), jnp.float32)], axis=0)
    rowi = lax.broadcasted_iota(jnp.int32, (_NOUT, 1), 0)
    dxyz = jnp.where(rowi < neff - 1, xf - xf_next, 0.0)
    lin1 = jnp.dot(xf, s1_ref[...], preferred_element_type=jnp.float32)
    lin2 = jnp.dot(dxyz, s1_ref[...], preferred_element_type=jnp.float32)
    d0 = jnp.dot(xf, g0_ref[...], preferred_element_type=jnp.float32)
    d1 = jnp.dot(xf, g1_ref[...], preferred_element_type=jnp.float32)
    d2 = jnp.dot(xf, g2_ref[...], preferred_element_type=jnp.float32)
    dist = jnp.sqrt(d0 * d0 + d1 * d1 + d2 * d2)
    o_ref[...] = jnp.concatenate([lin1, lin2, dist], axis=1)


def _run_features(xg, scal, meta16):
    return pl.pallas_call(
        _feat_body,
        out_shape=jax.ShapeDtypeStruct((_NOUT, 1196), jnp.float32),
    )(xg, scal, meta16, jnp.asarray(_S1), jnp.asarray(_G0),
      jnp.asarray(_G1), jnp.asarray(_G2))


def kernel(x):
    x2d = x.reshape(_T, _W)
    sums, scal = _run_scan(x2d)
    order, meta = _make_compact()(sums[:, 0], sums[:, 1], scal.reshape(64))
    xg = _run_gather(order, x2d.reshape(_T, 1, _W))
    out = _run_features(xg.reshape(_NOUT, _W), scal, meta.reshape(1, 16))
    return out[:100]


# speedup vs baseline: 5.6985x; 1.0112x over previous
import functools

import jax
import jax.numpy as jnp
import numpy as np
from jax import lax
from jax.experimental import pallas as pl
from jax.experimental.pallas import tpu as pltpu
from jax.experimental.pallas import tpu_sc as plsc

_T = 16384
_LM = 115
_W = 345
_BF = 512
_NOUT = 128


_HI, _HJ = np.triu_indices(21, k=1)
_PI, _PJ = np.triu_indices(25, k=1)
_LI, _LJ = np.triu_indices(20, k=1)


def _build_consts():
    s1 = np.zeros((258, 153), np.float32)
    for j in range(63):
        s1[j, j] = 1.0
    for q in range(50):
        s1[3 * (21 + q // 2) + q % 2, 63 + q] = 1.0
    for q in range(40):
        s1[3 * (46 + q // 2) + q % 2, 113 + q] = 1.0

    g = np.zeros((3, 258, 890), np.float32)
    for p, (a, b) in enumerate(zip(_HI, _HJ)):
        for c in range(3):
            g[c, 3 * a + c, p] += 1.0
            g[c, 3 * b + c, p] -= 1.0
    for p, (a, b) in enumerate(zip(_PI, _PJ)):
        for c in range(2):
            g[c, 3 * (21 + a) + c, 210 + p] += 1.0
            g[c, 3 * (21 + b) + c, 210 + p] -= 1.0
    for p, (a, b) in enumerate(zip(_LI, _LJ)):
        for c in range(2):
            g[c, 3 * (46 + a) + c, 510 + p] += 1.0
            g[c, 3 * (46 + b) + c, 510 + p] -= 1.0
    for p, (a, b) in enumerate(zip(_LI, _LJ)):
        for c in range(2):
            g[c, 3 * (66 + a) + c, 700 + p] += 1.0
            g[c, 3 * (66 + b) + c, 700 + p] -= 1.0
    return s1, g[0], g[1], g[2]


_S1, _G0, _G1, _G2 = _build_consts()


def _scan_body(x_ref, sums_ref, scal_ref):
    i = pl.program_id(0)
    xb = x_ref[...]
    xb = jnp.where(jnp.isnan(xb), jnp.float32(0.0), xb)
    col = lax.broadcasted_iota(jnp.int32, xb.shape, 1)
    lmask = (col >= 120) & (col < 183)
    rmask = col >= 282
    xl = jnp.where(lmask, xb, 0.0)
    xr = jnp.where(rmask, xb, 0.0)
    lsum = jnp.sum(xl, axis=1, keepdims=True)
    rsum = jnp.sum(xr, axis=1, keepdims=True)
    sums_ref[...] = jnp.concatenate([lsum, rsum], axis=1)
    lnz = jnp.sum(jnp.where(lmask & (xb != 0.0), 1.0, 0.0))
    rnz = jnp.sum(jnp.where(rmask & (xb != 0.0), 1.0, 0.0))
    nkl = jnp.sum(jnp.where(lsum != 0.0, 1.0, 0.0))
    nkr = jnp.sum(jnp.where(rsum != 0.0, 1.0, 0.0))
    ri = lax.broadcasted_iota(jnp.int32, (4, 16), 0)
    row = (jnp.where(ri == 0, lnz, 0.0) + jnp.where(ri == 1, rnz, 0.0)
           + jnp.where(ri == 2, nkl, 0.0) + jnp.where(ri == 3, nkr, 0.0))

    @pl.when(i == 0)
    def _():
        scal_ref[...] = jnp.zeros_like(scal_ref)

    scal_ref[...] += row


def _run_scan(x2d):
    return pl.pallas_call(
        _scan_body,
        grid=(_T // _BF,),
        in_specs=[pl.BlockSpec((_BF, _W), lambda i: (i, 0))],
        out_specs=[pl.BlockSpec((_BF, 2), lambda i: (i, 0)),
                   pl.BlockSpec((4, 16), lambda i: (0, 0))],
        out_shape=[jax.ShapeDtypeStruct((_T, 2), jnp.float32),
                   jax.ShapeDtypeStruct((4, 16), jnp.float32)],
    )(x2d)


@functools.cache
def _make_compact():
    mesh = plsc.VectorSubcoreMesh(core_axis_name="c", subcore_axis_name="s")
    return pl.kernel(
        _compact_body,
        mesh=mesh,
        compiler_params=pltpu.CompilerParams(needs_layout_passes=False),
        out_type=[jax.ShapeDtypeStruct((_NOUT,), jnp.int32),
                  jax.ShapeDtypeStruct((16,), jnp.int32)],
        scratch_types=[pltpu.VMEM((2 * _T,), jnp.float32),
                       pltpu.VMEM((64,), jnp.float32),
                       pltpu.VMEM((16,), jnp.int32),
                       pltpu.VMEM((16,), jnp.int32),
                       pltpu.VMEM((16,), jnp.int32),
                       pltpu.VMEM((16,), jnp.int32),
                       pltpu.VMEM((16,), jnp.int32),
                       pltpu.VMEM((_NOUT,), jnp.int32),
                       pltpu.VMEM((16,), jnp.int32)],
    )


def _compact_body(sums_hbm, scal_hbm, order_hbm, meta_hbm,
                  sums_v, sv, tmp_v, cond_r, nk_r, fi_r, kc_r,
                  order_v, meta_v):
    first = (lax.axis_index("c") == 0) & (lax.axis_index("s") == 0)

    @pl.when(first)
    def _():
        pltpu.sync_copy(sums_hbm, sums_v)
        pltpu.sync_copy(scal_hbm, sv)
        lane = lax.broadcasted_iota(jnp.int32, (16,), 0)
        lnzv = sv[pl.ds(0, 16)]
        rnzv = sv[pl.ds(16, 16)]
        cond_v = lnzv > rnzv
        nk_v = jnp.where(cond_v, sv[pl.ds(32, 16)],
                         sv[pl.ds(48, 16)]).astype(jnp.int32)
        neff_v = jnp.where(nk_v == 0, jnp.int32(_T), nk_v)
        cond_r[...] = cond_v.astype(jnp.int32)
        nk_r[...] = nk_v
        fi_r[...] = lane
        kc_r[...] = jnp.zeros((16,), jnp.int32)

        for j in range(_NOUT // 16):
            order_v[pl.ds(j * 16, 16)] = jnp.zeros((16,), jnp.int32)

        nk_s = nk_v[0]

        def loop_cond(carry):
            i, kc = carry
            drops = i * 16 - kc
            done = ((kc >= jnp.minimum(nk_s, _NOUT))
                    & ((nk_s >= _NOUT) | (drops >= _NOUT - nk_s)))
            return (i < _T // 16) & jnp.logical_not(done)

        def body_fn(carry):
            i, kc_sc = carry
            ln = lax.broadcasted_iota(jnp.int32, (16,), 0)
            condv = cond_r[...] != 0
            nkv = nk_r[...]
            fi = fi_r[...]
            kc = kc_r[...]
            fi2 = fi + fi
            lvv = plsc.load_gather(sums_v, [fi2])
            rvv = plsc.load_gather(sums_v, [fi2 + 1])
            selv = jnp.where(condv, lvv, rvv)
            kept = selv != 0.0
            ki = kept.astype(jnp.int32)
            cs = ki
            for k in (1, 2, 4, 8):
                tmp_v[...] = cs
                sh = plsc.load_gather(tmp_v, [jnp.maximum(ln - k, 0)],
                                      mask=ln >= k)
                cs = cs + jnp.where(ln >= k, sh, 0)
            kcv = kc + cs
            dest = jnp.where(kept, kcv - 1, nkv + fi - kcv)
            plsc.store_scatter(order_v, [dest], fi, mask=dest < _NOUT)
            tmp_v[...] = kcv
            kc_r[...] = plsc.load_gather(tmp_v, [jnp.full((16,), 15,
                                                          jnp.int32)])
            fi_r[...] = fi + 16
            return i + 1, kcv[15]

        lax.while_loop(loop_cond, body_fn, (jnp.int32(0), jnp.int32(0)))

        meta_v[...] = jnp.where(lane == 0, neff_v, 0)
        pltpu.sync_copy(order_v, order_hbm)
        pltpu.sync_copy(meta_v, meta_hbm)


def _gather_body(ord_ref, x_ref, o_ref):
    del ord_ref
    o_ref[...] = x_ref[...]


def _run_gather(order, x3):
    grid_spec = pltpu.PrefetchScalarGridSpec(
        num_scalar_prefetch=1,
        grid=(_NOUT,),
        in_specs=[pl.BlockSpec((1, 1, _W), lambda i, o: (o[i], 0, 0))],
        out_specs=pl.BlockSpec((1, 1, _W), lambda i, o: (i, 0, 0)),
    )
    return pl.pallas_call(
        _gather_body,
        grid_spec=grid_spec,
        out_shape=jax.ShapeDtypeStruct((_NOUT, 1, _W), jnp.float32),
    )(order, x3)


def _feat_body(xg_ref, scal_ref, meta_ref, s1_ref, g0_ref, g1_ref, g2_ref,
               o_ref):
    xg = xg_ref[...]
    xg = jnp.where(jnp.isnan(xg), jnp.float32(0.0), xg)
    cond = scal_ref[0, 0] > scal_ref[1, 0]
    neff = meta_ref[0, 0]
    hand = jnp.where(cond, xg[:, 120:183], xg[:, 282:345])
    xf = jnp.concatenate([hand, xg[:, 183:258], xg[:, 0:120]], axis=1)
    colmod = lax.broadcasted_iota(jnp.int32, (1, 258), 1) % 3
    xf = xf * jnp.where(cond & (colmod == 0), -1.0, 1.0)
    xf_next = jnp.concatenate(
        [xf[1:, :], jnp.zeros((1, 258), jnp.float32)], axis=0)
    rowi = lax.broadcasted_iota(jnp.int32, (_NOUT, 1), 0)
    dxyz = jnp.where(rowi < neff - 1, xf - xf_next, 0.0)
    lin1 = jnp.dot(xf, s1_ref[...], preferred_element_type=jnp.float32)
    lin2 = jnp.dot(dxyz, s1_ref[...], preferred_element_type=jnp.float32)
    d0 = jnp.dot(xf, g0_ref[...], preferred_element_type=jnp.float32)
    d1 = jnp.dot(xf, g1_ref[...], preferred_element_type=jnp.float32)
    d2 = jnp.dot(xf, g2_ref[...], preferred_element_type=jnp.float32)
    dist = jnp.sqrt(d0 * d0 + d1 * d1 + d2 * d2)
    o_ref[...] = jnp.concatenate([lin1, lin2, dist], axis=1)


def _run_features(xg, scal, meta16):
    return pl.pallas_call(
        _feat_body,
        out_shape=jax.ShapeDtypeStruct((_NOUT, 1196), jnp.float32),
    )(xg, scal, meta16, jnp.asarray(_S1), jnp.asarray(_G0),
      jnp.asarray(_G1), jnp.asarray(_G2))


def kernel(x):
    x2d = x.reshape(_T, _W)
    sums, scal = _run_scan(x2d)
    order, meta = _make_compact()(sums.reshape(2 * _T), scal.reshape(64))
    xg = _run_gather(order, x2d.reshape(_T, 1, _W))
    out = _run_features(xg.reshape(_NOUT, _W), scal, meta.reshape(1, 16))
    return out[:100]


# speedup vs baseline: 8.7837x; 1.5414x over previous
import functools

import jax
import jax.numpy as jnp
import numpy as np
from jax import lax
from jax.experimental import pallas as pl
from jax.experimental.pallas import tpu as pltpu
from jax.experimental.pallas import tpu_sc as plsc

_T = 16384
_LM = 115
_W = 345
_BF = 512
_NOUT = 128


_HI, _HJ = np.triu_indices(21, k=1)
_PI, _PJ = np.triu_indices(25, k=1)
_LI, _LJ = np.triu_indices(20, k=1)


def _build_consts():
    s1 = np.zeros((258, 153), np.float32)
    for j in range(63):
        s1[j, j] = 1.0
    for q in range(50):
        s1[3 * (21 + q // 2) + q % 2, 63 + q] = 1.0
    for q in range(40):
        s1[3 * (46 + q // 2) + q % 2, 113 + q] = 1.0

    g = np.zeros((3, 258, 890), np.float32)
    for p, (a, b) in enumerate(zip(_HI, _HJ)):
        for c in range(3):
            g[c, 3 * a + c, p] += 1.0
            g[c, 3 * b + c, p] -= 1.0
    for p, (a, b) in enumerate(zip(_PI, _PJ)):
        for c in range(2):
            g[c, 3 * (21 + a) + c, 210 + p] += 1.0
            g[c, 3 * (21 + b) + c, 210 + p] -= 1.0
    for p, (a, b) in enumerate(zip(_LI, _LJ)):
        for c in range(2):
            g[c, 3 * (46 + a) + c, 510 + p] += 1.0
            g[c, 3 * (46 + b) + c, 510 + p] -= 1.0
    for p, (a, b) in enumerate(zip(_LI, _LJ)):
        for c in range(2):
            g[c, 3 * (66 + a) + c, 700 + p] += 1.0
            g[c, 3 * (66 + b) + c, 700 + p] -= 1.0
    return s1, g[0], g[1], g[2]


_S1, _G0, _G1, _G2 = _build_consts()


def _scan_body(x_ref, sums_ref, scal_ref):
    i = pl.program_id(0)
    xb = x_ref[...]
    xb = jnp.where(jnp.isnan(xb), jnp.float32(0.0), xb)
    col = lax.broadcasted_iota(jnp.int32, xb.shape, 1)
    lmask = (col >= 120) & (col < 183)
    rmask = col >= 282
    xl = jnp.where(lmask, xb, 0.0)
    xr = jnp.where(rmask, xb, 0.0)
    lsum = jnp.sum(xl, axis=1, keepdims=True)
    rsum = jnp.sum(xr, axis=1, keepdims=True)
    sums_ref[...] = jnp.concatenate([lsum, rsum], axis=1)
    lnz = jnp.sum(jnp.where(lmask & (xb != 0.0), 1.0, 0.0))
    rnz = jnp.sum(jnp.where(rmask & (xb != 0.0), 1.0, 0.0))
    nkl = jnp.sum(jnp.where(lsum != 0.0, 1.0, 0.0))
    nkr = jnp.sum(jnp.where(rsum != 0.0, 1.0, 0.0))
    ri = lax.broadcasted_iota(jnp.int32, (4, 16), 0)
    row = (jnp.where(ri == 0, lnz, 0.0) + jnp.where(ri == 1, rnz, 0.0)
           + jnp.where(ri == 2, nkl, 0.0) + jnp.where(ri == 3, nkr, 0.0))

    @pl.when(i == 0)
    def _():
        scal_ref[...] = jnp.zeros_like(scal_ref)

    scal_ref[...] += row


def _run_scan(x2d):
    return pl.pallas_call(
        _scan_body,
        grid=(_T // _BF,),
        in_specs=[pl.BlockSpec((_BF, _W), lambda i: (i, 0))],
        out_specs=[pl.BlockSpec((_BF, 2), lambda i: (i, 0)),
                   pl.BlockSpec((4, 16), lambda i: (0, 0))],
        out_shape=[jax.ShapeDtypeStruct((_T, 2), jnp.float32),
                   jax.ShapeDtypeStruct((4, 16), jnp.float32)],
    )(x2d)


@functools.cache
def _make_compact():
    mesh = plsc.VectorSubcoreMesh(core_axis_name="c", subcore_axis_name="s")
    return pl.kernel(
        _compact_body,
        mesh=mesh,
        compiler_params=pltpu.CompilerParams(needs_layout_passes=False),
        out_type=[jax.ShapeDtypeStruct((_NOUT,), jnp.int32),
                  jax.ShapeDtypeStruct((16,), jnp.int32)],
        scratch_types=[pltpu.VMEM((2 * _T,), jnp.float32),
                       pltpu.VMEM((64,), jnp.float32),
                       pltpu.VMEM((16,), jnp.int32),
                       pltpu.VMEM((16,), jnp.int32),
                       pltpu.VMEM((16,), jnp.int32),
                       pltpu.VMEM((16,), jnp.int32),
                       pltpu.VMEM((16,), jnp.int32),
                       pltpu.VMEM((_NOUT,), jnp.int32),
                       pltpu.VMEM((16,), jnp.int32)],
    )


def _compact_body(sums_hbm, scal_hbm, order_hbm, meta_hbm,
                  sums_v, sv, tmp_v, cond_r, nk_r, fi_r, kc_r,
                  order_v, meta_v):
    first = (lax.axis_index("c") == 0) & (lax.axis_index("s") == 0)

    @pl.when(first)
    def _():
        pltpu.sync_copy(sums_hbm, sums_v)
        pltpu.sync_copy(scal_hbm, sv)
        lane = lax.broadcasted_iota(jnp.int32, (16,), 0)
        lnzv = sv[pl.ds(0, 16)]
        rnzv = sv[pl.ds(16, 16)]
        cond_v = lnzv > rnzv
        nk_v = jnp.where(cond_v, sv[pl.ds(32, 16)],
                         sv[pl.ds(48, 16)]).astype(jnp.int32)
        neff_v = jnp.where(nk_v == 0, jnp.int32(_T), nk_v)
        cond_r[...] = cond_v.astype(jnp.int32)
        nk_r[...] = nk_v
        fi_r[...] = lane
        kc_r[...] = jnp.zeros((16,), jnp.int32)

        for j in range(_NOUT // 16):
            order_v[pl.ds(j * 16, 16)] = jnp.zeros((16,), jnp.int32)

        nk_s = nk_v[0]

        def loop_cond(carry):
            i, kc = carry
            drops = i * 16 - kc
            done = ((kc >= jnp.minimum(nk_s, _NOUT))
                    & ((nk_s >= _NOUT) | (drops >= _NOUT - nk_s)))
            return (i < _T // 16) & jnp.logical_not(done)

        def body_fn(carry):
            i, kc_sc = carry
            ln = lax.broadcasted_iota(jnp.int32, (16,), 0)
            condv = cond_r[...] != 0
            nkv = nk_r[...]
            fi = fi_r[...]
            kc = kc_r[...]
            fi2 = fi + fi
            lvv = plsc.load_gather(sums_v, [fi2])
            rvv = plsc.load_gather(sums_v, [fi2 + 1])
            selv = jnp.where(condv, lvv, rvv)
            kept = selv != 0.0
            ki = kept.astype(jnp.int32)
            cs = ki
            for k in (1, 2, 4, 8):
                tmp_v[...] = cs
                sh = plsc.load_gather(tmp_v, [jnp.maximum(ln - k, 0)],
                                      mask=ln >= k)
                cs = cs + jnp.where(ln >= k, sh, 0)
            kcv = kc + cs
            dest = jnp.where(kept, kcv - 1, nkv + fi - kcv)
            plsc.store_scatter(order_v, [dest], fi, mask=dest < _NOUT)
            tmp_v[...] = kcv
            kc_r[...] = plsc.load_gather(tmp_v, [jnp.full((16,), 15,
                                                          jnp.int32)])
            fi_r[...] = fi + 16
            return i + 1, kcv[15]

        lax.while_loop(loop_cond, body_fn, (jnp.int32(0), jnp.int32(0)))

        meta_v[...] = jnp.where(lane == 0, neff_v, 0)
        pltpu.sync_copy(order_v, order_hbm)
        pltpu.sync_copy(meta_v, meta_hbm)


def _gather_body(ord_ref, x_ref, o_ref, sem):
    k = 16
    for c in range(_NOUT // k):
        for j in range(k):
            i = c * k + j
            pltpu.make_async_copy(
                x_ref.at[pl.ds(ord_ref[i], 1), :],
                o_ref.at[pl.ds(i, 1), :],
                sem,
            ).start()
        for j in range(k):
            i = c * k + j
            pltpu.make_async_copy(
                x_ref.at[pl.ds(ord_ref[i], 1), :],
                o_ref.at[pl.ds(i, 1), :],
                sem,
            ).wait()


def _run_gather(order, x2d):
    return pl.pallas_call(
        _gather_body,
        in_specs=[pl.BlockSpec(memory_space=pltpu.MemorySpace.SMEM),
                  pl.BlockSpec(memory_space=pltpu.MemorySpace.HBM)],
        out_specs=pl.BlockSpec((_NOUT, _W), lambda: (0, 0)),
        out_shape=jax.ShapeDtypeStruct((_NOUT, _W), jnp.float32),
        scratch_shapes=[pltpu.SemaphoreType.DMA],
    )(order, x2d)


def _feat_body(xg_ref, scal_ref, meta_ref, s1_ref, g0_ref, g1_ref, g2_ref,
               o_ref):
    xg = xg_ref[...]
    xg = jnp.where(jnp.isnan(xg), jnp.float32(0.0), xg)
    cond = scal_ref[0, 0] > scal_ref[1, 0]
    neff = meta_ref[0, 0]
    hand = jnp.where(cond, xg[:, 120:183], xg[:, 282:345])
    xf = jnp.concatenate([hand, xg[:, 183:258], xg[:, 0:120]], axis=1)
    colmod = lax.broadcasted_iota(jnp.int32, (1, 258), 1) % 3
    xf = xf * jnp.where(cond & (colmod == 0), -1.0, 1.0)
    xf_next = jnp.concatenate(
        [xf[1:, :], jnp.zeros((1, 258), jnp.float32)], axis=0)
    rowi = lax.broadcasted_iota(jnp.int32, (_NOUT, 1), 0)
    dxyz = jnp.where(rowi < neff - 1, xf - xf_next, 0.0)
    lin1 = jnp.dot(xf, s1_ref[...], preferred_element_type=jnp.float32)
    lin2 = jnp.dot(dxyz, s1_ref[...], preferred_element_type=jnp.float32)
    d0 = jnp.dot(xf, g0_ref[...], preferred_element_type=jnp.float32)
    d1 = jnp.dot(xf, g1_ref[...], preferred_element_type=jnp.float32)
    d2 = jnp.dot(xf, g2_ref[...], preferred_element_type=jnp.float32)
    dist = jnp.sqrt(d0 * d0 + d1 * d1 + d2 * d2)
    o_ref[...] = jnp.concatenate([lin1, lin2, dist], axis=1)


def _run_features(xg, scal, meta16):
    return pl.pallas_call(
        _feat_body,
        out_shape=jax.ShapeDtypeStruct((_NOUT, 1196), jnp.float32),
    )(xg, scal, meta16, jnp.asarray(_S1), jnp.asarray(_G0),
      jnp.asarray(_G1), jnp.asarray(_G2))


def kernel(x):
    x2d = x.reshape(_T, _W)
    sums, scal = _run_scan(x2d)
    order, meta = _make_compact()(sums.reshape(2 * _T), scal.reshape(64))
    xg = _run_gather(order, x2d)
    out = _run_features(xg, scal, meta.reshape(1, 16))
    return out[:100]


# speedup vs baseline: 9.5629x; 1.0887x over previous
import functools

import jax
import jax.numpy as jnp
import numpy as np
from jax import lax
from jax.experimental import pallas as pl
from jax.experimental.pallas import tpu as pltpu
from jax.experimental.pallas import tpu_sc as plsc

_T = 16384
_LM = 115
_W = 345
_BF = 512
_NOUT = 128


_HI, _HJ = np.triu_indices(21, k=1)
_PI, _PJ = np.triu_indices(25, k=1)
_LI, _LJ = np.triu_indices(20, k=1)


def _build_consts():
    s1 = np.zeros((258, 153), np.float32)
    for j in range(63):
        s1[j, j] = 1.0
    for q in range(50):
        s1[3 * (21 + q // 2) + q % 2, 63 + q] = 1.0
    for q in range(40):
        s1[3 * (46 + q // 2) + q % 2, 113 + q] = 1.0

    g = np.zeros((3, 258, 890), np.float32)
    for p, (a, b) in enumerate(zip(_HI, _HJ)):
        for c in range(3):
            g[c, 3 * a + c, p] += 1.0
            g[c, 3 * b + c, p] -= 1.0
    for p, (a, b) in enumerate(zip(_PI, _PJ)):
        for c in range(2):
            g[c, 3 * (21 + a) + c, 210 + p] += 1.0
            g[c, 3 * (21 + b) + c, 210 + p] -= 1.0
    for p, (a, b) in enumerate(zip(_LI, _LJ)):
        for c in range(2):
            g[c, 3 * (46 + a) + c, 510 + p] += 1.0
            g[c, 3 * (46 + b) + c, 510 + p] -= 1.0
    for p, (a, b) in enumerate(zip(_LI, _LJ)):
        for c in range(2):
            g[c, 3 * (66 + a) + c, 700 + p] += 1.0
            g[c, 3 * (66 + b) + c, 700 + p] -= 1.0
    return s1, g[0], g[1], g[2]


_S1, _G0, _G1, _G2 = _build_consts()
_S1T = np.ascontiguousarray(_S1.T)
_G0T = np.ascontiguousarray(_G0.T)
_G1T = np.ascontiguousarray(_G1.T)
_G2T = np.ascontiguousarray(_G2.T)


def _scan_body(x_ref, sums_ref, scal_ref):
    i = pl.program_id(0)
    xb = x_ref[...]
    xb = jnp.where(jnp.isnan(xb), jnp.float32(0.0), xb)
    row = lax.broadcasted_iota(jnp.int32, xb.shape, 0)
    lmask = (row >= 120) & (row < 183)
    rmask = row >= 282
    xl = jnp.where(lmask, xb, 0.0)
    xr = jnp.where(rmask, xb, 0.0)
    lsum = jnp.sum(xl, axis=0, keepdims=True)
    rsum = jnp.sum(xr, axis=0, keepdims=True)
    sums_ref[...] = jnp.concatenate([lsum, rsum], axis=0)
    lnz = jnp.sum(jnp.where(lmask & (xb != 0.0), 1.0, 0.0))
    rnz = jnp.sum(jnp.where(rmask & (xb != 0.0), 1.0, 0.0))
    nkl = jnp.sum(jnp.where(lsum != 0.0, 1.0, 0.0))
    nkr = jnp.sum(jnp.where(rsum != 0.0, 1.0, 0.0))
    ri = lax.broadcasted_iota(jnp.int32, (4, 16), 0)
    srow = (jnp.where(ri == 0, lnz, 0.0) + jnp.where(ri == 1, rnz, 0.0)
            + jnp.where(ri == 2, nkl, 0.0) + jnp.where(ri == 3, nkr, 0.0))

    @pl.when(i == 0)
    def _():
        scal_ref[...] = jnp.zeros_like(scal_ref)

    scal_ref[...] += srow


def _run_scan(xt):
    return pl.pallas_call(
        _scan_body,
        grid=(_T // _BF,),
        in_specs=[pl.BlockSpec((_W, _BF), lambda i: (0, i))],
        out_specs=[pl.BlockSpec((2, _BF), lambda i: (0, i)),
                   pl.BlockSpec((4, 16), lambda i: (0, 0))],
        out_shape=[jax.ShapeDtypeStruct((2, _T), jnp.float32),
                   jax.ShapeDtypeStruct((4, 16), jnp.float32)],
    )(xt)


@functools.cache
def _make_compact():
    mesh = plsc.VectorSubcoreMesh(core_axis_name="c", subcore_axis_name="s")
    return pl.kernel(
        _compact_body,
        mesh=mesh,
        compiler_params=pltpu.CompilerParams(needs_layout_passes=False),
        out_type=[jax.ShapeDtypeStruct((_NOUT,), jnp.int32),
                  jax.ShapeDtypeStruct((16,), jnp.int32)],
        scratch_types=[pltpu.VMEM((2 * _T,), jnp.float32),
                       pltpu.VMEM((64,), jnp.float32),
                       pltpu.VMEM((16,), jnp.int32),
                       pltpu.VMEM((16,), jnp.int32),
                       pltpu.VMEM((16,), jnp.int32),
                       pltpu.VMEM((16,), jnp.int32),
                       pltpu.VMEM((16,), jnp.int32),
                       pltpu.VMEM((_NOUT,), jnp.int32),
                       pltpu.VMEM((16,), jnp.int32)],
    )


def _compact_body(sums_hbm, scal_hbm, order_hbm, meta_hbm,
                  sums_v, sv, tmp_v, cond_r, nk_r, fi_r, kc_r,
                  order_v, meta_v):
    first = (lax.axis_index("c") == 0) & (lax.axis_index("s") == 0)

    @pl.when(first)
    def _():
        pltpu.sync_copy(sums_hbm, sums_v)
        pltpu.sync_copy(scal_hbm, sv)
        lane = lax.broadcasted_iota(jnp.int32, (16,), 0)
        lnzv = sv[pl.ds(0, 16)]
        rnzv = sv[pl.ds(16, 16)]
        cond_v = lnzv > rnzv
        nk_v = jnp.where(cond_v, sv[pl.ds(32, 16)],
                         sv[pl.ds(48, 16)]).astype(jnp.int32)
        neff_v = jnp.where(nk_v == 0, jnp.int32(_T), nk_v)
        cond_r[...] = cond_v.astype(jnp.int32)
        nk_r[...] = nk_v
        fi_r[...] = lane
        kc_r[...] = jnp.zeros((16,), jnp.int32)

        for j in range(_NOUT // 16):
            order_v[pl.ds(j * 16, 16)] = jnp.zeros((16,), jnp.int32)

        nk_s = nk_v[0]

        def loop_cond(carry):
            i, kc = carry
            drops = i * 16 - kc
            done = ((kc >= jnp.minimum(nk_s, _NOUT))
                    & ((nk_s >= _NOUT) | (drops >= _NOUT - nk_s)))
            return (i < _T // 16) & jnp.logical_not(done)

        def body_fn(carry):
            i, kc_sc = carry
            ln = lax.broadcasted_iota(jnp.int32, (16,), 0)
            condv = cond_r[...] != 0
            nkv = nk_r[...]
            fi = fi_r[...]
            kc = kc_r[...]
            lvv = sums_v[pl.ds(i * 16, 16)]
            rvv = sums_v[pl.ds(_T + i * 16, 16)]
            selv = jnp.where(condv, lvv, rvv)
            kept = selv != 0.0
            ki = kept.astype(jnp.int32)
            cs = ki
            for k in (1, 2, 4, 8):
                tmp_v[...] = cs
                sh = plsc.load_gather(tmp_v, [jnp.maximum(ln - k, 0)],
                                      mask=ln >= k)
                cs = cs + jnp.where(ln >= k, sh, 0)
            kcv = kc + cs
            dest = jnp.where(kept, kcv - 1, nkv + fi - kcv)
            plsc.store_scatter(order_v, [dest], fi, mask=dest < _NOUT)
            tmp_v[...] = kcv
            kc_r[...] = plsc.load_gather(tmp_v, [jnp.full((16,), 15,
                                                          jnp.int32)])
            fi_r[...] = fi + 16
            return i + 1, kcv[15]

        lax.while_loop(loop_cond, body_fn, (jnp.int32(0), jnp.int32(0)))

        meta_v[...] = jnp.where(lane == 0, neff_v, 0)
        pltpu.sync_copy(order_v, order_hbm)
        pltpu.sync_copy(meta_v, meta_hbm)


def _feat_body(xt_ref, ord_ref, scal_ref, meta_ref, s1_ref, g0_ref, g1_ref,
               g2_ref, o_ref, xg_ref):
    i = pl.program_id(0)
    xb = xt_ref[...]
    xb = jnp.where(jnp.isnan(xb), jnp.float32(0.0), xb)
    fid = lax.broadcasted_iota(jnp.int32, (_BF, _NOUT), 0) + i * _BF
    p = jnp.where(fid == ord_ref[...], 1.0, 0.0)
    acc = jnp.dot(xb, p, preferred_element_type=jnp.float32)

    @pl.when(i == 0)
    def _():
        xg_ref[...] = jnp.zeros_like(xg_ref)

    xg_ref[...] += acc

    @pl.when(i == _T // _BF - 1)
    def _():
        xg = xg_ref[...]
        cond = scal_ref[0, 0] > scal_ref[1, 0]
        neff = meta_ref[0, 0]
        hand = jnp.where(cond, xg[120:183, :], xg[282:345, :])
        xf = jnp.concatenate([hand, xg[183:258, :], xg[0:120, :]], axis=0)
        rowmod = lax.broadcasted_iota(jnp.int32, (258, 1), 0) % 3
        xf = xf * jnp.where(cond & (rowmod == 0), -1.0, 1.0)
        xf_next = jnp.concatenate(
            [xf[:, 1:], jnp.zeros((258, 1), jnp.float32)], axis=1)
        coli = lax.broadcasted_iota(jnp.int32, (1, _NOUT), 1)
        dxyz = jnp.where(coli < neff - 1, xf - xf_next, 0.0)
        lin1 = jnp.dot(s1_ref[...], xf, preferred_element_type=jnp.float32)
        lin2 = jnp.dot(s1_ref[...], dxyz,
                       preferred_element_type=jnp.float32)
        d0 = jnp.dot(g0_ref[...], xf, preferred_element_type=jnp.float32)
        d1 = jnp.dot(g1_ref[...], xf, preferred_element_type=jnp.float32)
        d2 = jnp.dot(g2_ref[...], xf, preferred_element_type=jnp.float32)
        dist = jnp.sqrt(d0 * d0 + d1 * d1 + d2 * d2)
        o_ref[...] = jnp.concatenate([lin1, lin2, dist], axis=0)


def _run_features(xt, order, scal, meta16):
    return pl.pallas_call(
        _feat_body,
        grid=(_T // _BF,),
        in_specs=[pl.BlockSpec((_W, _BF), lambda i: (0, i)),
                  pl.BlockSpec((1, _NOUT), lambda i: (0, 0)),
                  pl.BlockSpec((4, 16), lambda i: (0, 0)),
                  pl.BlockSpec((1, 16), lambda i: (0, 0)),
                  pl.BlockSpec((153, 258), lambda i: (0, 0)),
                  pl.BlockSpec((890, 258), lambda i: (0, 0)),
                  pl.BlockSpec((890, 258), lambda i: (0, 0)),
                  pl.BlockSpec((890, 258), lambda i: (0, 0))],
        out_specs=pl.BlockSpec((1196, _NOUT), lambda i: (0, 0)),
        out_shape=jax.ShapeDtypeStruct((1196, _NOUT), jnp.float32),
        scratch_shapes=[pltpu.VMEM((_W, _NOUT), jnp.float32)],
    )(xt, order, scal, meta16, jnp.asarray(_S1T), jnp.asarray(_G0T),
      jnp.asarray(_G1T), jnp.asarray(_G2T))


def kernel(x):
    xt = x.transpose(1, 2, 0).reshape(_W, _T)
    sums, scal = _run_scan(xt)
    order, meta = _make_compact()(sums.reshape(2 * _T), scal.reshape(64))
    outt = _run_features(xt, order.reshape(1, _NOUT), scal,
                         meta.reshape(1, 16))
    return outt.T[:100]


# speedup vs baseline: 19.9354x; 2.0847x over previous
import functools

import jax
import jax.numpy as jnp
import numpy as np
from jax import lax
from jax.experimental import pallas as pl
from jax.experimental.pallas import tpu as pltpu
from jax.experimental.pallas import tpu_sc as plsc

_T = 16384
_LM = 115
_W = 345
_BF = 512
_NOUT = 128


_HI, _HJ = np.triu_indices(21, k=1)
_PI, _PJ = np.triu_indices(25, k=1)
_LI, _LJ = np.triu_indices(20, k=1)


def _build_consts():
    def col(l, c):
        return c * 86 + l

    s1 = np.zeros((258, 153), np.float32)
    for j in range(63):
        s1[col(j // 3, j % 3), j] = 1.0
    for q in range(50):
        s1[col(21 + q // 2, q % 2), 63 + q] = 1.0
    for q in range(40):
        s1[col(46 + q // 2, q % 2), 113 + q] = 1.0

    g = np.zeros((3, 258, 890), np.float32)
    for p, (a, b) in enumerate(zip(_HI, _HJ)):
        for c in range(3):
            g[c, col(a, c), p] += 1.0
            g[c, col(b, c), p] -= 1.0
    for p, (a, b) in enumerate(zip(_PI, _PJ)):
        for c in range(2):
            g[c, col(21 + a, c), 210 + p] += 1.0
            g[c, col(21 + b, c), 210 + p] -= 1.0
    for p, (a, b) in enumerate(zip(_LI, _LJ)):
        for c in range(2):
            g[c, col(46 + a, c), 510 + p] += 1.0
            g[c, col(46 + b, c), 510 + p] -= 1.0
    for p, (a, b) in enumerate(zip(_LI, _LJ)):
        for c in range(2):
            g[c, col(66 + a, c), 700 + p] += 1.0
            g[c, col(66 + b, c), 700 + p] -= 1.0
    return s1, g[0], g[1], g[2]


_S1, _G0, _G1, _G2 = _build_consts()
_S1T = np.ascontiguousarray(_S1.T)
_G0T = np.ascontiguousarray(_G0.T)
_G1T = np.ascontiguousarray(_G1.T)
_G2T = np.ascontiguousarray(_G2.T)


def _scan_body(x_ref, sums_ref, scal_ref):
    i = pl.program_id(0)
    xb = x_ref[...]
    xb = jnp.where(jnp.isnan(xb), jnp.float32(0.0), xb)
    rl = lax.broadcasted_iota(jnp.int32, xb.shape, 0) % 115
    lmask = (rl >= 40) & (rl < 61)
    rmask = rl >= 94
    xl = jnp.where(lmask, xb, 0.0)
    xr = jnp.where(rmask, xb, 0.0)
    lsum = jnp.sum(xl, axis=0, keepdims=True)
    rsum = jnp.sum(xr, axis=0, keepdims=True)
    sums_ref[...] = jnp.concatenate([lsum, rsum], axis=0)
    lnz = jnp.sum(jnp.where(lmask & (xb != 0.0), 1.0, 0.0))
    rnz = jnp.sum(jnp.where(rmask & (xb != 0.0), 1.0, 0.0))
    nkl = jnp.sum(jnp.where(lsum != 0.0, 1.0, 0.0))
    nkr = jnp.sum(jnp.where(rsum != 0.0, 1.0, 0.0))
    ri = lax.broadcasted_iota(jnp.int32, (4, 16), 0)
    srow = (jnp.where(ri == 0, lnz, 0.0) + jnp.where(ri == 1, rnz, 0.0)
            + jnp.where(ri == 2, nkl, 0.0) + jnp.where(ri == 3, nkr, 0.0))

    @pl.when(i == 0)
    def _():
        scal_ref[...] = jnp.zeros_like(scal_ref)

    scal_ref[...] += srow


def _run_scan(xt):
    return pl.pallas_call(
        _scan_body,
        grid=(_T // _BF,),
        in_specs=[pl.BlockSpec((_W, _BF), lambda i: (0, i))],
        out_specs=[pl.BlockSpec((2, _BF), lambda i: (0, i)),
                   pl.BlockSpec((4, 16), lambda i: (0, 0))],
        out_shape=[jax.ShapeDtypeStruct((2, _T), jnp.float32),
                   jax.ShapeDtypeStruct((4, 16), jnp.float32)],
    )(xt)


@functools.cache
def _make_compact():
    mesh = plsc.VectorSubcoreMesh(core_axis_name="c", subcore_axis_name="s")
    return pl.kernel(
        _compact_body,
        mesh=mesh,
        compiler_params=pltpu.CompilerParams(needs_layout_passes=False),
        out_type=[jax.ShapeDtypeStruct((_NOUT,), jnp.int32),
                  jax.ShapeDtypeStruct((16,), jnp.int32)],
        scratch_types=[pltpu.VMEM((2 * _T,), jnp.float32),
                       pltpu.VMEM((64,), jnp.float32),
                       pltpu.VMEM((16,), jnp.int32),
                       pltpu.VMEM((16,), jnp.int32),
                       pltpu.VMEM((16,), jnp.int32),
                       pltpu.VMEM((16,), jnp.int32),
                       pltpu.VMEM((16,), jnp.int32),
                       pltpu.VMEM((_NOUT,), jnp.int32),
                       pltpu.VMEM((16,), jnp.int32)],
    )


def _compact_body(sums_hbm, scal_hbm, order_hbm, meta_hbm,
                  sums_v, sv, tmp_v, cond_r, nk_r, fi_r, kc_r,
                  order_v, meta_v):
    first = (lax.axis_index("c") == 0) & (lax.axis_index("s") == 0)

    @pl.when(first)
    def _():
        pltpu.sync_copy(sums_hbm, sums_v)
        pltpu.sync_copy(scal_hbm, sv)
        lane = lax.broadcasted_iota(jnp.int32, (16,), 0)
        lnzv = sv[pl.ds(0, 16)]
        rnzv = sv[pl.ds(16, 16)]
        cond_v = lnzv > rnzv
        nk_v = jnp.where(cond_v, sv[pl.ds(32, 16)],
                         sv[pl.ds(48, 16)]).astype(jnp.int32)
        neff_v = jnp.where(nk_v == 0, jnp.int32(_T), nk_v)
        cond_r[...] = cond_v.astype(jnp.int32)
        nk_r[...] = nk_v
        fi_r[...] = lane
        kc_r[...] = jnp.zeros((16,), jnp.int32)

        for j in range(_NOUT // 16):
            order_v[pl.ds(j * 16, 16)] = jnp.zeros((16,), jnp.int32)

        nk_s = nk_v[0]

        def loop_cond(carry):
            i, kc = carry
            drops = i * 16 - kc
            done = ((kc >= jnp.minimum(nk_s, _NOUT))
                    & ((nk_s >= _NOUT) | (drops >= _NOUT - nk_s)))
            return (i < _T // 16) & jnp.logical_not(done)

        def body_fn(carry):
            i, kc_sc = carry
            ln = lax.broadcasted_iota(jnp.int32, (16,), 0)
            condv = cond_r[...] != 0
            nkv = nk_r[...]
            fi = fi_r[...]
            kc = kc_r[...]
            lvv = sums_v[pl.ds(i * 16, 16)]
            rvv = sums_v[pl.ds(_T + i * 16, 16)]
            selv = jnp.where(condv, lvv, rvv)
            kept = selv != 0.0
            ki = kept.astype(jnp.int32)
            cs = ki
            for k in (1, 2, 4, 8):
                tmp_v[...] = cs
                sh = plsc.load_gather(tmp_v, [jnp.maximum(ln - k, 0)],
                                      mask=ln >= k)
                cs = cs + jnp.where(ln >= k, sh, 0)
            kcv = kc + cs
            dest = jnp.where(kept, kcv - 1, nkv + fi - kcv)
            plsc.store_scatter(order_v, [dest], fi, mask=dest < _NOUT)
            tmp_v[...] = kcv
            kc_r[...] = plsc.load_gather(tmp_v, [jnp.full((16,), 15,
                                                          jnp.int32)])
            fi_r[...] = fi + 16
            return i + 1, kcv[15]

        lax.while_loop(loop_cond, body_fn, (jnp.int32(0), jnp.int32(0)))

        meta_v[...] = jnp.where(lane == 0, neff_v, 0)
        pltpu.sync_copy(order_v, order_hbm)
        pltpu.sync_copy(meta_v, meta_hbm)


def _feat_body(xt_ref, ord_ref, scal_ref, meta_ref, s1_ref, g0_ref, g1_ref,
               g2_ref, o_ref, xg_ref):
    i = pl.program_id(0)
    xb = xt_ref[...]
    xb = jnp.where(jnp.isnan(xb), jnp.float32(0.0), xb)
    fid = lax.broadcasted_iota(jnp.int32, (_BF, _NOUT), 0) + i * _BF
    p = jnp.where(fid == ord_ref[...], 1.0, 0.0)
    acc = jnp.dot(xb, p, preferred_element_type=jnp.float32)

    @pl.when(i == 0)
    def _():
        xg_ref[...] = jnp.zeros_like(xg_ref)

    xg_ref[...] += acc

    @pl.when(i == _T // _BF - 1)
    def _():
        xg = xg_ref[...]
        cond = scal_ref[0, 0] > scal_ref[1, 0]
        neff = meta_ref[0, 0]
        parts = []
        for c in range(3):
            b = c * 115
            hand_c = jnp.where(cond, xg[b + 40:b + 61, :],
                               xg[b + 94:b + 115, :])
            parts += [hand_c, xg[b + 61:b + 86, :], xg[b:b + 40, :]]
        xf = jnp.concatenate(parts, axis=0)
        rowc = lax.broadcasted_iota(jnp.int32, (258, 1), 0)
        xf = xf * jnp.where(cond & (rowc < 86), -1.0, 1.0)
        xf_next = jnp.concatenate(
            [xf[:, 1:], jnp.zeros((258, 1), jnp.float32)], axis=1)
        coli = lax.broadcasted_iota(jnp.int32, (1, _NOUT), 1)
        dxyz = jnp.where(coli < neff - 1, xf - xf_next, 0.0)
        lin1 = jnp.dot(s1_ref[...], xf, preferred_element_type=jnp.float32)
        lin2 = jnp.dot(s1_ref[...], dxyz,
                       preferred_element_type=jnp.float32)
        d0 = jnp.dot(g0_ref[...], xf, preferred_element_type=jnp.float32)
        d1 = jnp.dot(g1_ref[...], xf, preferred_element_type=jnp.float32)
        d2 = jnp.dot(g2_ref[...], xf, preferred_element_type=jnp.float32)
        dist = jnp.sqrt(d0 * d0 + d1 * d1 + d2 * d2)
        o_ref[...] = jnp.concatenate([lin1, lin2, dist], axis=0)


def _run_features(xt, order, scal, meta16):
    return pl.pallas_call(
        _feat_body,
        grid=(_T // _BF,),
        in_specs=[pl.BlockSpec((_W, _BF), lambda i: (0, i)),
                  pl.BlockSpec((1, _NOUT), lambda i: (0, 0)),
                  pl.BlockSpec((4, 16), lambda i: (0, 0)),
                  pl.BlockSpec((1, 16), lambda i: (0, 0)),
                  pl.BlockSpec((153, 258), lambda i: (0, 0)),
                  pl.BlockSpec((890, 258), lambda i: (0, 0)),
                  pl.BlockSpec((890, 258), lambda i: (0, 0)),
                  pl.BlockSpec((890, 258), lambda i: (0, 0))],
        out_specs=pl.BlockSpec((1196, _NOUT), lambda i: (0, 0)),
        out_shape=jax.ShapeDtypeStruct((1196, _NOUT), jnp.float32),
        scratch_shapes=[pltpu.VMEM((_W, _NOUT), jnp.float32)],
    )(xt, order, scal, meta16, jnp.asarray(_S1T), jnp.asarray(_G0T),
      jnp.asarray(_G1T), jnp.asarray(_G2T))


def kernel(x):
    xt = x.transpose(2, 1, 0).reshape(_W, _T)
    sums, scal = _run_scan(xt)
    order, meta = _make_compact()(sums.reshape(2 * _T), scal.reshape(64))
    outt = _run_features(xt, order.reshape(1, _NOUT), scal,
                         meta.reshape(1, 16))
    return outt.T[:100]


# speedup vs baseline: 25.0213x; 1.2551x over previous
import functools

import jax
import jax.numpy as jnp
import numpy as np
from jax import lax
from jax.experimental import pallas as pl
from jax.experimental.pallas import tpu as pltpu
from jax.experimental.pallas import tpu_sc as plsc

_T = 16384
_LM = 115
_W = 345
_BF = 512
_NOUT = 128


_HI, _HJ = np.triu_indices(21, k=1)
_PI, _PJ = np.triu_indices(25, k=1)
_LI, _LJ = np.triu_indices(20, k=1)


def _build_consts():
    def col(l, c):
        return c * 86 + l

    s1 = np.zeros((258, 153), np.float32)
    for j in range(63):
        s1[col(j // 3, j % 3), j] = 1.0
    for q in range(50):
        s1[col(21 + q // 2, q % 2), 63 + q] = 1.0
    for q in range(40):
        s1[col(46 + q // 2, q % 2), 113 + q] = 1.0

    g = np.zeros((3, 258, 890), np.float32)
    for p, (a, b) in enumerate(zip(_HI, _HJ)):
        for c in range(3):
            g[c, col(a, c), p] += 1.0
            g[c, col(b, c), p] -= 1.0
    for p, (a, b) in enumerate(zip(_PI, _PJ)):
        for c in range(2):
            g[c, col(21 + a, c), 210 + p] += 1.0
            g[c, col(21 + b, c), 210 + p] -= 1.0
    for p, (a, b) in enumerate(zip(_LI, _LJ)):
        for c in range(2):
            g[c, col(46 + a, c), 510 + p] += 1.0
            g[c, col(46 + b, c), 510 + p] -= 1.0
    for p, (a, b) in enumerate(zip(_LI, _LJ)):
        for c in range(2):
            g[c, col(66 + a, c), 700 + p] += 1.0
            g[c, col(66 + b, c), 700 + p] -= 1.0
    return s1, g[0], g[1], g[2]


_S1, _G0, _G1, _G2 = _build_consts()
_S1T = np.ascontiguousarray(_S1.T)
_G0T = np.ascontiguousarray(_G0.T)
_G1T = np.ascontiguousarray(_G1.T)
_G2T = np.ascontiguousarray(_G2.T)


def _scan_body(x_ref, sums_ref, scal_ref):
    i = pl.program_id(0)
    xb = x_ref[...]
    xb = jnp.where(jnp.isnan(xb), jnp.float32(0.0), xb)
    rl = lax.broadcasted_iota(jnp.int32, xb.shape, 0) % 115
    lmask = (rl >= 40) & (rl < 61)
    rmask = rl >= 94
    xl = jnp.where(lmask, xb, 0.0)
    xr = jnp.where(rmask, xb, 0.0)
    lsum = jnp.sum(xl, axis=0, keepdims=True)
    rsum = jnp.sum(xr, axis=0, keepdims=True)
    sums_ref[...] = jnp.concatenate([lsum, rsum], axis=0)
    lnz = jnp.sum(jnp.where(lmask & (xb != 0.0), 1.0, 0.0))
    rnz = jnp.sum(jnp.where(rmask & (xb != 0.0), 1.0, 0.0))
    nkl = jnp.sum(jnp.where(lsum != 0.0, 1.0, 0.0))
    nkr = jnp.sum(jnp.where(rsum != 0.0, 1.0, 0.0))
    ri = lax.broadcasted_iota(jnp.int32, (4, 16), 0)
    srow = (jnp.where(ri == 0, lnz, 0.0) + jnp.where(ri == 1, rnz, 0.0)
            + jnp.where(ri == 2, nkl, 0.0) + jnp.where(ri == 3, nkr, 0.0))

    @pl.when(i == 0)
    def _():
        scal_ref[...] = jnp.zeros_like(scal_ref)

    scal_ref[...] += srow


def _run_scan(xt):
    return pl.pallas_call(
        _scan_body,
        grid=(_T // _BF,),
        in_specs=[pl.BlockSpec((_W, _BF), lambda i: (0, i))],
        out_specs=[pl.BlockSpec((2, _BF), lambda i: (0, i)),
                   pl.BlockSpec((4, 16), lambda i: (0, 0))],
        out_shape=[jax.ShapeDtypeStruct((2, _T), jnp.float32),
                   jax.ShapeDtypeStruct((4, 16), jnp.float32)],
    )(xt)


@functools.cache
def _make_compact():
    mesh = plsc.VectorSubcoreMesh(core_axis_name="c", subcore_axis_name="s")
    return pl.kernel(
        _compact_body,
        mesh=mesh,
        compiler_params=pltpu.CompilerParams(needs_layout_passes=False),
        out_type=[jax.ShapeDtypeStruct((_NOUT,), jnp.int32),
                  jax.ShapeDtypeStruct((16,), jnp.int32),
                  jax.ShapeDtypeStruct((_T // _BF,), jnp.int32)],
        scratch_types=[pltpu.VMEM((2 * _T,), jnp.float32),
                       pltpu.VMEM((64,), jnp.float32),
                       pltpu.VMEM((16,), jnp.int32),
                       pltpu.VMEM((16,), jnp.int32),
                       pltpu.VMEM((16,), jnp.int32),
                       pltpu.VMEM((16,), jnp.int32),
                       pltpu.VMEM((16,), jnp.int32),
                       pltpu.VMEM((_NOUT,), jnp.int32),
                       pltpu.VMEM((16,), jnp.int32),
                       pltpu.VMEM((_T // _BF,), jnp.int32)],
    )


def _compact_body(sums_hbm, scal_hbm, order_hbm, meta_hbm, blk_hbm,
                  sums_v, sv, tmp_v, cond_r, nk_r, fi_r, kc_r,
                  order_v, meta_v, blk_v):
    first = (lax.axis_index("c") == 0) & (lax.axis_index("s") == 0)

    @pl.when(first)
    def _():
        pltpu.sync_copy(sums_hbm, sums_v)
        pltpu.sync_copy(scal_hbm, sv)
        lane = lax.broadcasted_iota(jnp.int32, (16,), 0)
        lnzv = sv[pl.ds(0, 16)]
        rnzv = sv[pl.ds(16, 16)]
        cond_v = lnzv > rnzv
        nk_v = jnp.where(cond_v, sv[pl.ds(32, 16)],
                         sv[pl.ds(48, 16)]).astype(jnp.int32)
        neff_v = jnp.where(nk_v == 0, jnp.int32(_T), nk_v)
        cond_r[...] = cond_v.astype(jnp.int32)
        nk_r[...] = nk_v
        fi_r[...] = lane
        kc_r[...] = jnp.zeros((16,), jnp.int32)

        for j in range(_NOUT // 16):
            order_v[pl.ds(j * 16, 16)] = jnp.zeros((16,), jnp.int32)

        nk_s = nk_v[0]

        def loop_cond(carry):
            i, kc = carry
            drops = i * 16 - kc
            done = ((kc >= jnp.minimum(nk_s, _NOUT))
                    & ((nk_s >= _NOUT) | (drops >= _NOUT - nk_s)))
            return (i < _T // 16) & jnp.logical_not(done)

        def body_fn(carry):
            i, kc_sc = carry
            ln = lax.broadcasted_iota(jnp.int32, (16,), 0)
            condv = cond_r[...] != 0
            nkv = nk_r[...]
            fi = fi_r[...]
            kc = kc_r[...]
            lvv = sums_v[pl.ds(i * 16, 16)]
            rvv = sums_v[pl.ds(_T + i * 16, 16)]
            selv = jnp.where(condv, lvv, rvv)
            kept = selv != 0.0
            ki = kept.astype(jnp.int32)
            cs = ki
            for k in (1, 2, 4, 8):
                tmp_v[...] = cs
                sh = plsc.load_gather(tmp_v, [jnp.maximum(ln - k, 0)],
                                      mask=ln >= k)
                cs = cs + jnp.where(ln >= k, sh, 0)
            kcv = kc + cs
            dest = jnp.where(kept, kcv - 1, nkv + fi - kcv)
            plsc.store_scatter(order_v, [dest], fi, mask=dest < _NOUT)
            tmp_v[...] = kcv
            kc_r[...] = plsc.load_gather(tmp_v, [jnp.full((16,), 15,
                                                          jnp.int32)])
            fi_r[...] = fi + 16
            return i + 1, kcv[15]

        lax.while_loop(loop_cond, body_fn, (jnp.int32(0), jnp.int32(0)))

        meta_v[...] = jnp.where(lane == 0, neff_v, 0)
        for j in range(2):
            blk_v[pl.ds(j * 16, 16)] = jnp.zeros((16,), jnp.int32)
        ones = jnp.full((16,), 1, jnp.int32)
        for j in range(_NOUT // 16):
            ids = order_v[pl.ds(j * 16, 16)]
            plsc.store_scatter(blk_v, [lax.shift_right_logical(ids, 9)],
                               ones)
        pltpu.sync_copy(order_v, order_hbm)
        pltpu.sync_copy(meta_v, meta_hbm)
        pltpu.sync_copy(blk_v, blk_hbm)


def _feat_body(blk_ref, ord_ref, xt_ref, scal_ref, meta_ref, s1_ref,
               g0_ref, g1_ref, g2_ref, o_ref, xg_ref, xb_ref, sem):
    i = pl.program_id(0)

    @pl.when(i == 0)
    def _():
        xg_ref[...] = jnp.zeros_like(xg_ref)

    @pl.when(blk_ref[i] != 0)
    def _():
        cp = pltpu.make_async_copy(
            xt_ref.at[:, pl.ds(i * _BF, _BF)], xb_ref, sem)
        cp.start()
        cp.wait()
        xb = xb_ref[...]
        xb = jnp.where(jnp.isnan(xb), jnp.float32(0.0), xb)
        fid = lax.broadcasted_iota(jnp.int32, (_BF, _NOUT), 0) + i * _BF
        p = jnp.where(fid == ord_ref[...], 1.0, 0.0)
        xg_ref[...] += jnp.dot(xb, p, preferred_element_type=jnp.float32)

    @pl.when(i == _T // _BF - 1)
    def _():
        _feat_tail(scal_ref, meta_ref, s1_ref, g0_ref, g1_ref, g2_ref,
                   o_ref, xg_ref)


def _feat_tail(scal_ref, meta_ref, s1_ref, g0_ref, g1_ref, g2_ref,
               o_ref, xg_ref):
    xg = xg_ref[...]
    cond = scal_ref[0, 0] > scal_ref[1, 0]
    neff = meta_ref[0, 0]
    parts = []
    for c in range(3):
        b = c * 115
        hand_c = jnp.where(cond, xg[b + 40:b + 61, :],
                           xg[b + 94:b + 115, :])
        parts += [hand_c, xg[b + 61:b + 86, :], xg[b:b + 40, :]]
    xf = jnp.concatenate(parts, axis=0)
    rowc = lax.broadcasted_iota(jnp.int32, (258, 1), 0)
    xf = xf * jnp.where(cond & (rowc < 86), -1.0, 1.0)
    xf_next = jnp.concatenate(
        [xf[:, 1:], jnp.zeros((258, 1), jnp.float32)], axis=1)
    coli = lax.broadcasted_iota(jnp.int32, (1, _NOUT), 1)
    dxyz = jnp.where(coli < neff - 1, xf - xf_next, 0.0)
    lin1 = jnp.dot(s1_ref[...], xf, preferred_element_type=jnp.float32)
    lin2 = jnp.dot(s1_ref[...], dxyz, preferred_element_type=jnp.float32)
    d0 = jnp.dot(g0_ref[...], xf, preferred_element_type=jnp.float32)
    d1 = jnp.dot(g1_ref[...], xf, preferred_element_type=jnp.float32)
    d2 = jnp.dot(g2_ref[...], xf, preferred_element_type=jnp.float32)
    dist = jnp.sqrt(d0 * d0 + d1 * d1 + d2 * d2)
    o_ref[...] = jnp.concatenate([lin1, lin2, dist], axis=0)


def _run_features(xt, blk, order, scal, meta16):
    nb = _T // _BF
    return pl.pallas_call(
        _feat_body,
        grid=(nb,),
        in_specs=[pl.BlockSpec(memory_space=pltpu.MemorySpace.SMEM),
                  pl.BlockSpec((1, _NOUT), lambda i: (0, 0)),
                  pl.BlockSpec(memory_space=pltpu.MemorySpace.HBM),
                  pl.BlockSpec((4, 16), lambda i: (0, 0)),
                  pl.BlockSpec((1, 16), lambda i: (0, 0)),
                  pl.BlockSpec((153, 258), lambda i: (0, 0)),
                  pl.BlockSpec((890, 258), lambda i: (0, 0)),
                  pl.BlockSpec((890, 258), lambda i: (0, 0)),
                  pl.BlockSpec((890, 258), lambda i: (0, 0))],
        out_specs=pl.BlockSpec((1196, _NOUT), lambda i: (0, 0)),
        out_shape=jax.ShapeDtypeStruct((1196, _NOUT), jnp.float32),
        scratch_shapes=[pltpu.VMEM((_W, _NOUT), jnp.float32),
                        pltpu.VMEM((_W, _BF), jnp.float32),
                        pltpu.SemaphoreType.DMA],
    )(blk, order, xt, scal, meta16, jnp.asarray(_S1T), jnp.asarray(_G0T),
      jnp.asarray(_G1T), jnp.asarray(_G2T))


def kernel(x):
    xt = x.transpose(2, 1, 0).reshape(_W, _T)
    sums, scal = _run_scan(xt)
    order, meta, blk = _make_compact()(sums.reshape(2 * _T),
                                       scal.reshape(64))
    outt = _run_features(xt, blk, order.reshape(1, _NOUT), scal,
                         meta.reshape(1, 16))
    return outt.T[:100]


# speedup vs baseline: 27.0768x; 1.0822x over previous
import functools

import jax
import jax.numpy as jnp
import numpy as np
from jax import lax
from jax.experimental import pallas as pl
from jax.experimental.pallas import tpu as pltpu
from jax.experimental.pallas import tpu_sc as plsc

_T = 16384
_LM = 115
_W = 345
_BF = 512
_NOUT = 128


_HI, _HJ = np.triu_indices(21, k=1)
_PI, _PJ = np.triu_indices(25, k=1)
_LI, _LJ = np.triu_indices(20, k=1)


def _build_consts():
    def col(l, c):
        return c * 86 + l

    s1 = np.zeros((258, 153), np.float32)
    for j in range(63):
        s1[col(j // 3, j % 3), j] = 1.0
    for q in range(50):
        s1[col(21 + q // 2, q % 2), 63 + q] = 1.0
    for q in range(40):
        s1[col(46 + q // 2, q % 2), 113 + q] = 1.0

    g = np.zeros((3, 258, 890), np.float32)
    for p, (a, b) in enumerate(zip(_HI, _HJ)):
        for c in range(3):
            g[c, col(a, c), p] += 1.0
            g[c, col(b, c), p] -= 1.0
    for p, (a, b) in enumerate(zip(_PI, _PJ)):
        for c in range(2):
            g[c, col(21 + a, c), 210 + p] += 1.0
            g[c, col(21 + b, c), 210 + p] -= 1.0
    for p, (a, b) in enumerate(zip(_LI, _LJ)):
        for c in range(2):
            g[c, col(46 + a, c), 510 + p] += 1.0
            g[c, col(46 + b, c), 510 + p] -= 1.0
    for p, (a, b) in enumerate(zip(_LI, _LJ)):
        for c in range(2):
            g[c, col(66 + a, c), 700 + p] += 1.0
            g[c, col(66 + b, c), 700 + p] -= 1.0
    return s1, g[0], g[1], g[2]


_S1, _G0, _G1, _G2 = _build_consts()
_S1T = np.ascontiguousarray(_S1.T)
_G0T = np.ascontiguousarray(_G0.T)
_G1T = np.ascontiguousarray(_G1.T)
_G2T = np.ascontiguousarray(_G2.T)


_MASKS = np.zeros((2, _W), np.float32)
_MASKS[0, :] = ((np.arange(_W) % 115 >= 40) & (np.arange(_W) % 115 < 61))
_MASKS[1, :] = (np.arange(_W) % 115 >= 94)


def _scan_body(x_ref, m_ref, sums_ref, scal_ref, acc_ref):
    i = pl.program_id(0)
    xb = x_ref[...]
    xb = jnp.where(jnp.isnan(xb), jnp.float32(0.0), xb)
    z = jnp.where(xb != 0.0, 1.0, 0.0)
    m = m_ref[...]
    s2 = jnp.dot(m, xb, preferred_element_type=jnp.float32)
    c2 = jnp.dot(m, z, preferred_element_type=jnp.float32)
    sums_ref[...] = s2
    k2 = jnp.where(s2 != 0.0, 1.0, 0.0)

    @pl.when(i == 0)
    def _():
        acc_ref[...] = jnp.zeros_like(acc_ref)

    acc_ref[...] += jnp.concatenate([c2, k2], axis=0)

    @pl.when(i == _T // _BF - 1)
    def _():
        tot = jnp.sum(acc_ref[...], axis=1, keepdims=True)
        scal_ref[...] = jnp.broadcast_to(tot, (4, 16))


def _run_scan(xt):
    return pl.pallas_call(
        _scan_body,
        grid=(_T // _BF,),
        in_specs=[pl.BlockSpec((_W, _BF), lambda i: (0, i)),
                  pl.BlockSpec((2, _W), lambda i: (0, 0))],
        out_specs=[pl.BlockSpec((2, _BF), lambda i: (0, i)),
                   pl.BlockSpec((4, 16), lambda i: (0, 0))],
        out_shape=[jax.ShapeDtypeStruct((2, _T), jnp.float32),
                   jax.ShapeDtypeStruct((4, 16), jnp.float32)],
        scratch_shapes=[pltpu.VMEM((4, _BF), jnp.float32)],
    )(xt, jnp.asarray(_MASKS))


@functools.cache
def _make_compact():
    mesh = plsc.VectorSubcoreMesh(core_axis_name="c", subcore_axis_name="s")
    return pl.kernel(
        _compact_body,
        mesh=mesh,
        compiler_params=pltpu.CompilerParams(needs_layout_passes=False),
        out_type=[jax.ShapeDtypeStruct((_NOUT,), jnp.int32),
                  jax.ShapeDtypeStruct((16,), jnp.int32),
                  jax.ShapeDtypeStruct((_T // _BF,), jnp.int32)],
        scratch_types=[pltpu.VMEM((2 * _T,), jnp.float32),
                       pltpu.VMEM((64,), jnp.float32),
                       pltpu.VMEM((16,), jnp.int32),
                       pltpu.VMEM((16,), jnp.int32),
                       pltpu.VMEM((16,), jnp.int32),
                       pltpu.VMEM((16,), jnp.int32),
                       pltpu.VMEM((16,), jnp.int32),
                       pltpu.VMEM((_NOUT,), jnp.int32),
                       pltpu.VMEM((16,), jnp.int32),
                       pltpu.VMEM((_T // _BF,), jnp.int32)],
    )


def _compact_body(sums_hbm, scal_hbm, order_hbm, meta_hbm, blk_hbm,
                  sums_v, sv, tmp_v, cond_r, nk_r, fi_r, kc_r,
                  order_v, meta_v, blk_v):
    first = (lax.axis_index("c") == 0) & (lax.axis_index("s") == 0)

    @pl.when(first)
    def _():
        pltpu.sync_copy(sums_hbm, sums_v)
        pltpu.sync_copy(scal_hbm, sv)
        lane = lax.broadcasted_iota(jnp.int32, (16,), 0)
        lnzv = sv[pl.ds(0, 16)]
        rnzv = sv[pl.ds(16, 16)]
        cond_v = lnzv > rnzv
        nk_v = jnp.where(cond_v, sv[pl.ds(32, 16)],
                         sv[pl.ds(48, 16)]).astype(jnp.int32)
        neff_v = jnp.where(nk_v == 0, jnp.int32(_T), nk_v)
        cond_r[...] = cond_v.astype(jnp.int32)
        nk_r[...] = nk_v
        fi_r[...] = lane
        kc_r[...] = jnp.zeros((16,), jnp.int32)

        for j in range(_NOUT // 16):
            order_v[pl.ds(j * 16, 16)] = jnp.zeros((16,), jnp.int32)

        nk_s = nk_v[0]

        def loop_cond(carry):
            i, kc = carry
            drops = i * 16 - kc
            done = ((kc >= jnp.minimum(nk_s, _NOUT))
                    & ((nk_s >= _NOUT) | (drops >= _NOUT - nk_s)))
            return (i < _T // 16) & jnp.logical_not(done)

        def body_fn(carry):
            i, kc_sc = carry
            ln = lax.broadcasted_iota(jnp.int32, (16,), 0)
            condv = cond_r[...] != 0
            nkv = nk_r[...]
            fi = fi_r[...]
            kc = kc_r[...]
            lvv = sums_v[pl.ds(i * 16, 16)]
            rvv = sums_v[pl.ds(_T + i * 16, 16)]
            selv = jnp.where(condv, lvv, rvv)
            kept = selv != 0.0
            ki = kept.astype(jnp.int32)
            cs = ki
            for k in (1, 2, 4, 8):
                tmp_v[...] = cs
                sh = plsc.load_gather(tmp_v, [jnp.maximum(ln - k, 0)],
                                      mask=ln >= k)
                cs = cs + jnp.where(ln >= k, sh, 0)
            kcv = kc + cs
            dest = jnp.where(kept, kcv - 1, nkv + fi - kcv)
            plsc.store_scatter(order_v, [dest], fi, mask=dest < _NOUT)
            tmp_v[...] = kcv
            kc_r[...] = plsc.load_gather(tmp_v, [jnp.full((16,), 15,
                                                          jnp.int32)])
            fi_r[...] = fi + 16
            return i + 1, kcv[15]

        lax.while_loop(loop_cond, body_fn, (jnp.int32(0), jnp.int32(0)))

        meta_v[...] = jnp.where(lane == 0, neff_v, 0)
        for j in range(2):
            blk_v[pl.ds(j * 16, 16)] = jnp.zeros((16,), jnp.int32)
        ones = jnp.full((16,), 1, jnp.int32)
        for j in range(_NOUT // 16):
            ids = order_v[pl.ds(j * 16, 16)]
            plsc.store_scatter(blk_v, [lax.shift_right_logical(ids, 9)],
                               ones)
        pltpu.sync_copy(order_v, order_hbm)
        pltpu.sync_copy(meta_v, meta_hbm)
        pltpu.sync_copy(blk_v, blk_hbm)


def _feat_body(blk_ref, ord_ref, xt_ref, scal_ref, meta_ref, s1_ref,
               g0_ref, g1_ref, g2_ref, o_ref, xg_ref, xb_ref, sem):
    i = pl.program_id(0)

    @pl.when(i == 0)
    def _():
        xg_ref[...] = jnp.zeros_like(xg_ref)

    @pl.when(blk_ref[i] != 0)
    def _():
        cp = pltpu.make_async_copy(
            xt_ref.at[:, pl.ds(i * _BF, _BF)], xb_ref, sem)
        cp.start()
        cp.wait()
        xb = xb_ref[...]
        xb = jnp.where(jnp.isnan(xb), jnp.float32(0.0), xb)
        fid = lax.broadcasted_iota(jnp.int32, (_BF, _NOUT), 0) + i * _BF
        p = jnp.where(fid == ord_ref[...], 1.0, 0.0)
        xg_ref[...] += jnp.dot(xb, p, preferred_element_type=jnp.float32)

    @pl.when(i == _T // _BF - 1)
    def _():
        _feat_tail(scal_ref, meta_ref, s1_ref, g0_ref, g1_ref, g2_ref,
                   o_ref, xg_ref)


def _feat_tail(scal_ref, meta_ref, s1_ref, g0_ref, g1_ref, g2_ref,
               o_ref, xg_ref):
    xg = xg_ref[...]
    cond = scal_ref[0, 0] > scal_ref[1, 0]
    neff = meta_ref[0, 0]
    parts = []
    for c in range(3):
        b = c * 115
        hand_c = jnp.where(cond, xg[b + 40:b + 61, :],
                           xg[b + 94:b + 115, :])
        parts += [hand_c, xg[b + 61:b + 86, :], xg[b:b + 40, :]]
    xf = jnp.concatenate(parts, axis=0)
    rowc = lax.broadcasted_iota(jnp.int32, (258, 1), 0)
    xf = xf * jnp.where(cond & (rowc < 86), -1.0, 1.0)
    xf_next = jnp.concatenate(
        [xf[:, 1:], jnp.zeros((258, 1), jnp.float32)], axis=1)
    coli = lax.broadcasted_iota(jnp.int32, (1, _NOUT), 1)
    dxyz = jnp.where(coli < neff - 1, xf - xf_next, 0.0)
    lin1 = jnp.dot(s1_ref[...], xf, preferred_element_type=jnp.float32)
    lin2 = jnp.dot(s1_ref[...], dxyz, preferred_element_type=jnp.float32)
    d0 = jnp.dot(g0_ref[...], xf, preferred_element_type=jnp.float32)
    d1 = jnp.dot(g1_ref[...], xf, preferred_element_type=jnp.float32)
    d2 = jnp.dot(g2_ref[...], xf, preferred_element_type=jnp.float32)
    dist = jnp.sqrt(d0 * d0 + d1 * d1 + d2 * d2)
    o_ref[...] = jnp.concatenate([lin1, lin2, dist], axis=0)


def _run_features(xt, blk, order, scal, meta16):
    nb = _T // _BF
    return pl.pallas_call(
        _feat_body,
        grid=(nb,),
        in_specs=[pl.BlockSpec(memory_space=pltpu.MemorySpace.SMEM),
                  pl.BlockSpec((1, _NOUT), lambda i: (0, 0)),
                  pl.BlockSpec(memory_space=pltpu.MemorySpace.HBM),
                  pl.BlockSpec((4, 16), lambda i: (0, 0)),
                  pl.BlockSpec((1, 16), lambda i: (0, 0)),
                  pl.BlockSpec((153, 258), lambda i: (0, 0)),
                  pl.BlockSpec((890, 258), lambda i: (0, 0)),
                  pl.BlockSpec((890, 258), lambda i: (0, 0)),
                  pl.BlockSpec((890, 258), lambda i: (0, 0))],
        out_specs=pl.BlockSpec((1196, _NOUT), lambda i: (0, 0)),
        out_shape=jax.ShapeDtypeStruct((1196, _NOUT), jnp.float32),
        scratch_shapes=[pltpu.VMEM((_W, _NOUT), jnp.float32),
                        pltpu.VMEM((_W, _BF), jnp.float32),
                        pltpu.SemaphoreType.DMA],
    )(blk, order, xt, scal, meta16, jnp.asarray(_S1T), jnp.asarray(_G0T),
      jnp.asarray(_G1T), jnp.asarray(_G2T))


def kernel(x):
    xt = x.transpose(2, 1, 0).reshape(_W, _T)
    sums, scal = _run_scan(xt)
    order, meta, blk = _make_compact()(sums.reshape(2 * _T),
                                       scal.reshape(64))
    outt = _run_features(xt, blk, order.reshape(1, _NOUT), scal,
                         meta.reshape(1, 16))
    return outt.T[:100]
